# baseline (speedup 1.0000x reference)
import functools

import jax
import jax.numpy as jnp
from jax.experimental import pallas as pl
from jax.experimental.pallas import tpu as pltpu

_F = 64
_EPS = 1e-5


def _rstd0(x):
    return jax.lax.rsqrt(jnp.mean(x * x, axis=0, keepdims=True) + _EPS)


def _ln_c(x, g, b):
    return x * _rstd0(x) * g + b


def _ln_f(x, g, b):
    m = jnp.mean(x, axis=0, keepdims=True)
    xc = x - m
    return xc * _rstd0(xc) * g + b


def _body(th_ref, chx_ref, chy_ref, cwx_ref, cwy_ref, sh_ref, sw_ref, inds_ref,
          pe_w1_ref, pe_b1_ref, pe_w2_ref,
          wavat_ref, wct_t_ref, wbvbt_ref, wbvb_c_ref,
          w_g_ref, w_b_ref, w_w2_ref, w_b2_ref,
          v_g_ref, v_b_ref, v_w2_ref, v_b2_ref,
          norm_g_ref, norm_b_ref, lin_w_ref, lin_g_ref, lin_b_ref,
          out_ref, *, th_sz, tw_sz):
    w = pl.program_id(2)
    n_w = pl.num_programs(2)

    @pl.when(w == 0)
    def _():
        out_ref[...] = jnp.zeros_like(out_ref)

    f32 = jnp.float32
    bf16 = jnp.bfloat16
    p = tw_sz * th_sz
    tcx = chx_ref[0]
    tcy = chy_ref[0]
    scx = cwx_ref[0]
    scy = cwy_ref[0]
    dx = (tcx - scx).reshape(1, p)
    dy = (tcy - scy).reshape(1, p)
    dist = jnp.sqrt(dx * dx + dy * dy)
    mask = (dist < th_ref[0, 0]).astype(f32)

    d_xy = jnp.concatenate([dx, dy], axis=0).astype(bf16)
    pe_h = jax.nn.relu(jnp.dot(pe_w1_ref[...], d_xy,
                               preferred_element_type=f32)
                       + pe_b1_ref[...])
    pe_y = jnp.dot(pe_w2_ref[...], pe_h.astype(bf16),
                   preferred_element_type=f32)
    pe_out = (pe_y * _rstd0(pe_y)).astype(bf16)

    src_h = sh_ref[0]
    src_w = sw_ref[0]
    pre_hc = jnp.dot(wct_t_ref[...], src_h, preferred_element_type=f32)
    pre = (jnp.dot(wavat_ref[...], src_w, preferred_element_type=f32)
           + wbvb_c_ref[...])
    pre_pair = jnp.broadcast_to(pre.reshape(2 * _F, tw_sz, 1),
                                (2 * _F, tw_sz, th_sz)).reshape(2 * _F, p)

    mixed = (jnp.dot(wbvbt_ref[...], pe_out, preferred_element_type=f32)
             + pre_pair)

    def ln_relu_bf16(xc, g_ref, b_ref):
        t = (xc * _rstd0(xc)).astype(bf16)
        return jax.nn.relu(t * g_ref[...].astype(bf16) + b_ref[...].astype(bf16))

    h1 = ln_relu_bf16(mixed[:_F] + pltpu.repeat(pre_hc, tw_sz, axis=1),
                      w_g_ref, w_b_ref)
    hv = ln_relu_bf16(mixed[_F:], v_g_ref, v_b_ref)
    wgt = jax.nn.sigmoid(jnp.dot(w_w2_ref[...], h1,
                                 preferred_element_type=f32) + w_b2_ref[...])
    val = jnp.dot(v_w2_ref[...], hv,
                  preferred_element_type=f32) + v_b2_ref[...]
    contrib = (wgt * val) * mask
    q = p
    while q > th_sz:
        q //= 2
        contrib = contrib[:, :q] + contrib[:, q:]
    out_ref[0] += contrib

    @pl.when(w == n_w - 1)
    def _():
        acc = out_ref[0]
        o = jax.nn.relu(_ln_f(acc, norm_g_ref[...], norm_b_ref[...]))
        o = _ln_c(jnp.dot(lin_w_ref[...], o, preferred_element_type=f32),
                  lin_g_ref[...], lin_b_ref[...])
        out_ref[0] = jax.nn.relu(o + src_h)


def kernel(src, src_coords, pe_w1, pe_b1, pe_w2, pe_ln_g, pe_ln_b,
           w_w1, w_ln_g, w_ln_b, w_w2, w_b2,
           v_w1, v_ln_g, v_ln_b, v_w2, v_b2,
           norm_g, norm_b, lin_w, lin_ln_g, lin_ln_b, dist_th):
    B, N, n = src.shape
    assert n == _F
    th_sz, tw_sz = 128, 128
    n_h, n_w = N // th_sz, N // tw_sz
    p = th_sz * tw_sz

    f32 = jnp.float32
    bf16 = jnp.bfloat16
    col = lambda a: a.reshape(-1, 1).astype(f32)
    th = jnp.asarray(dist_th, f32).reshape(1, 1)

    src_t = src.transpose(0, 2, 1)
    cx_row = src_coords[:, :, 0].reshape(B, 1, N)
    cy_row = src_coords[:, :, 1].reshape(B, 1, N)
    cx_col = src_coords[:, :, 0:1]
    cy_col = src_coords[:, :, 1:2]

    q_ids = jnp.arange(p, dtype=jnp.int32)
    inds = (jnp.arange(tw_sz, dtype=jnp.int32)[:, None]
            == (q_ids // th_sz)[None, :]).astype(bf16)

    def center_halves(a):
        top = a[:_F] - a[:_F].mean(axis=0, keepdims=True)
        bot = a[_F:] - a[_F:].mean(axis=0, keepdims=True)
        return jnp.concatenate([top, bot], axis=0)

    pe_w2 = pe_w2 - pe_w2.mean(axis=0, keepdims=True)
    lin_w = lin_w - lin_w.mean(axis=0, keepdims=True)
    wavat = center_halves(
        jnp.concatenate([w_w1[:, :_F], v_w1[:, :_F]], axis=0))
    wct_t = w_w1[:, 2 * _F:]
    wct_t = wct_t - wct_t.mean(axis=0, keepdims=True)
    wbvbt = jnp.concatenate([w_w1[:, _F:2 * _F], v_w1[:, _F:]], axis=0)
    wbvb_c = center_halves((wbvbt @ pe_ln_b).reshape(-1, 1))
    wbvbt = center_halves(wbvbt * pe_ln_g.reshape(1, _F))

    weight_args = (
        pe_w1.astype(bf16), col(pe_b1), pe_w2.astype(bf16),
        wavat, wct_t, wbvbt.astype(bf16), wbvb_c,
        col(w_ln_g), col(w_ln_b), w_w2.astype(bf16), col(w_b2),
        col(v_ln_g), col(v_ln_b), v_w2.astype(bf16), col(v_b2),
        col(norm_g), col(norm_b), lin_w, col(lin_ln_g), col(lin_ln_b),
    )

    def const_spec(a):
        nd = a.ndim
        return pl.BlockSpec(a.shape, lambda b, h, w, _nd=nd: (0,) * _nd)

    body = functools.partial(_body, th_sz=th_sz, tw_sz=tw_sz)

    in_specs = [
        const_spec(th),
        pl.BlockSpec((1, 1, th_sz), lambda b, h, w: (b, 0, h)),
        pl.BlockSpec((1, 1, th_sz), lambda b, h, w: (b, 0, h)),
        pl.BlockSpec((1, tw_sz, 1), lambda b, h, w: (b, w, 0)),
        pl.BlockSpec((1, tw_sz, 1), lambda b, h, w: (b, w, 0)),
        pl.BlockSpec((1, _F, th_sz), lambda b, h, w: (b, 0, h)),
        pl.BlockSpec((1, _F, tw_sz), lambda b, h, w: (b, 0, w)),
        const_spec(inds),
    ] + [const_spec(a) for a in weight_args]

    out_t = pl.pallas_call(
        body,
        grid=(B, n_h, n_w),
        in_specs=in_specs,
        out_specs=pl.BlockSpec((1, _F, th_sz), lambda b, h, w: (b, 0, h)),
        out_shape=jax.ShapeDtypeStruct((B, _F, N), f32),
        compiler_params=pltpu.CompilerParams(
            dimension_semantics=("parallel", "parallel", "arbitrary")),
    )(th, cx_row, cy_row, cx_col, cy_col, src_t, src_t, inds, *weight_args)
    return out_t.transpose(0, 2, 1)


# speedup vs baseline: 1.2640x; 1.2640x over previous
import functools

import jax
import jax.numpy as jnp
from jax.experimental import pallas as pl
from jax.experimental.pallas import tpu as pltpu

_F = 64
_EPS = 1e-5


def _rstd0(x):
    return jax.lax.rsqrt(jnp.mean(x * x, axis=0, keepdims=True) + _EPS)


def _ln_c(x, g, b):
    return x * _rstd0(x) * g + b


def _ln_f(x, g, b):
    m = jnp.mean(x, axis=0, keepdims=True)
    xc = x - m
    return xc * _rstd0(xc) * g + b


def _body(th_ref, chx_ref, chy_ref, cwx_ref, cwy_ref, sh_ref, sw_ref, inds_ref,
          pe_w1_ref, pe_b1_ref, pe_w2_ref,
          wavat_ref, wct_t_ref, wbvbt_ref, wbvb_c_ref,
          w_g_ref, w_b_ref, w_w2_ref, w_b2_ref,
          v_g_ref, v_b_ref, v_w2_ref, v_b2_ref,
          norm_g_ref, norm_b_ref, lin_w_ref, lin_g_ref, lin_b_ref,
          out_ref, s_ref, *, th_sz, tw_sz):
    w = pl.program_id(2)
    n_w = pl.num_programs(2)

    @pl.when(w == 0)
    def _():
        s_ref[_F:, :] = inds_ref[...]

    @pl.when(w == 0)
    def _():
        out_ref[...] = jnp.zeros_like(out_ref)

    f32 = jnp.float32
    bf16 = jnp.bfloat16
    p = tw_sz * th_sz
    tcx = chx_ref[0]
    tcy = chy_ref[0]
    scx = cwx_ref[0]
    scy = cwy_ref[0]
    dx = (tcx - scx).reshape(1, p)
    dy = (tcy - scy).reshape(1, p)
    dist = jnp.sqrt(dx * dx + dy * dy)
    mask = (dist < th_ref[0, 0]).astype(f32)

    d_xy = jnp.concatenate([dx, dy], axis=0).astype(bf16)
    pe_h = jax.nn.relu(jnp.dot(pe_w1_ref[...], d_xy,
                               preferred_element_type=f32)
                       + pe_b1_ref[...])
    pe_y = jnp.dot(pe_w2_ref[...], pe_h.astype(bf16),
                   preferred_element_type=f32)
    s_ref[:_F, :] = (pe_y * _rstd0(pe_y)).astype(bf16)

    src_h = sh_ref[0]
    src_w = sw_ref[0]
    pre_hc = jnp.dot(wct_t_ref[...], src_h, preferred_element_type=f32)
    pre = (jnp.dot(wavat_ref[...], src_w, preferred_element_type=f32)
           + wbvb_c_ref[...]).astype(bf16)

    wcat = jnp.concatenate([wbvbt_ref[...], pre], axis=1)
    mixed = jnp.dot(wcat, s_ref[...], preferred_element_type=f32)

    def ln_relu_bf16(xc, g_ref, b_ref):
        t = (xc * _rstd0(xc)).astype(bf16)
        return jax.nn.relu(t * g_ref[...].astype(bf16) + b_ref[...].astype(bf16))

    h1 = ln_relu_bf16(mixed[:_F] + pltpu.repeat(pre_hc, tw_sz, axis=1),
                      w_g_ref, w_b_ref)
    hv = ln_relu_bf16(mixed[_F:], v_g_ref, v_b_ref)
    wgt = jax.nn.sigmoid(jnp.dot(w_w2_ref[...], h1,
                                 preferred_element_type=f32) + w_b2_ref[...])
    val = jnp.dot(v_w2_ref[...], hv,
                  preferred_element_type=f32) + v_b2_ref[...]
    contrib = (wgt * val) * mask
    q = p
    while q > th_sz:
        q //= 2
        contrib = contrib[:, :q] + contrib[:, q:]
    out_ref[0] += contrib

    @pl.when(w == n_w - 1)
    def _():
        acc = out_ref[0]
        o = jax.nn.relu(_ln_f(acc, norm_g_ref[...], norm_b_ref[...]))
        o = _ln_c(jnp.dot(lin_w_ref[...], o, preferred_element_type=f32),
                  lin_g_ref[...], lin_b_ref[...])
        out_ref[0] = jax.nn.relu(o + src_h)


def kernel(src, src_coords, pe_w1, pe_b1, pe_w2, pe_ln_g, pe_ln_b,
           w_w1, w_ln_g, w_ln_b, w_w2, w_b2,
           v_w1, v_ln_g, v_ln_b, v_w2, v_b2,
           norm_g, norm_b, lin_w, lin_ln_g, lin_ln_b, dist_th):
    B, N, n = src.shape
    assert n == _F
    th_sz, tw_sz = 128, 128
    n_h, n_w = N // th_sz, N // tw_sz
    p = th_sz * tw_sz

    f32 = jnp.float32
    bf16 = jnp.bfloat16
    col = lambda a: a.reshape(-1, 1).astype(f32)
    th = jnp.asarray(dist_th, f32).reshape(1, 1)

    src_t = src.transpose(0, 2, 1)
    cx_row = src_coords[:, :, 0].reshape(B, 1, N)
    cy_row = src_coords[:, :, 1].reshape(B, 1, N)
    cx_col = src_coords[:, :, 0:1]
    cy_col = src_coords[:, :, 1:2]

    q_ids = jnp.arange(p, dtype=jnp.int32)
    inds = (jnp.arange(tw_sz, dtype=jnp.int32)[:, None]
            == (q_ids // th_sz)[None, :]).astype(bf16)

    def center_halves(a):
        top = a[:_F] - a[:_F].mean(axis=0, keepdims=True)
        bot = a[_F:] - a[_F:].mean(axis=0, keepdims=True)
        return jnp.concatenate([top, bot], axis=0)

    pe_w2 = pe_w2 - pe_w2.mean(axis=0, keepdims=True)
    lin_w = lin_w - lin_w.mean(axis=0, keepdims=True)
    wavat = center_halves(
        jnp.concatenate([w_w1[:, :_F], v_w1[:, :_F]], axis=0))
    wct_t = w_w1[:, 2 * _F:]
    wct_t = wct_t - wct_t.mean(axis=0, keepdims=True)
    wbvbt = jnp.concatenate([w_w1[:, _F:2 * _F], v_w1[:, _F:]], axis=0)
    wbvb_c = center_halves((wbvbt @ pe_ln_b).reshape(-1, 1))
    wbvbt = center_halves(wbvbt * pe_ln_g.reshape(1, _F))

    weight_args = (
        pe_w1.astype(bf16), col(pe_b1), pe_w2.astype(bf16),
        wavat, wct_t, wbvbt.astype(bf16), wbvb_c,
        col(w_ln_g), col(w_ln_b), w_w2.astype(bf16), col(w_b2),
        col(v_ln_g), col(v_ln_b), v_w2.astype(bf16), col(v_b2),
        col(norm_g), col(norm_b), lin_w, col(lin_ln_g), col(lin_ln_b),
    )

    def const_spec(a):
        nd = a.ndim
        return pl.BlockSpec(a.shape, lambda b, h, w, _nd=nd: (0,) * _nd)

    body = functools.partial(_body, th_sz=th_sz, tw_sz=tw_sz)

    in_specs = [
        const_spec(th),
        pl.BlockSpec((1, 1, th_sz), lambda b, h, w: (b, 0, h)),
        pl.BlockSpec((1, 1, th_sz), lambda b, h, w: (b, 0, h)),
        pl.BlockSpec((1, tw_sz, 1), lambda b, h, w: (b, w, 0)),
        pl.BlockSpec((1, tw_sz, 1), lambda b, h, w: (b, w, 0)),
        pl.BlockSpec((1, _F, th_sz), lambda b, h, w: (b, 0, h)),
        pl.BlockSpec((1, _F, tw_sz), lambda b, h, w: (b, 0, w)),
        const_spec(inds),
    ] + [const_spec(a) for a in weight_args]

    out_t = pl.pallas_call(
        body,
        grid=(B, n_h, n_w),
        in_specs=in_specs,
        out_specs=pl.BlockSpec((1, _F, th_sz), lambda b, h, w: (b, 0, h)),
        out_shape=jax.ShapeDtypeStruct((B, _F, N), f32),
        scratch_shapes=[pltpu.VMEM((3 * _F, p), bf16)],
        compiler_params=pltpu.CompilerParams(
            dimension_semantics=("parallel", "parallel", "arbitrary")),
    )(th, cx_row, cy_row, cx_col, cy_col, src_t, src_t, inds, *weight_args)
    return out_t.transpose(0, 2, 1)
